# Initial kernel scaffold; baseline (speedup 1.0000x reference)
import jax
import jax.numpy as jnp
from jax.experimental import pallas as pl


def kernel(offsets, W1, b1, W2, b2):
    raise NotImplementedError("write your pallas kernel here")


# speedup vs baseline: 10.3570x; 10.3570x over previous
import jax
import jax.numpy as jnp
from jax.experimental import pallas as pl

_RES = 64
_NVOX = _RES ** 3
_HID = 128
_OCC_THRE = 0.01

_BLK = 2048
_GRID = _NVOX // _BLK


def _bf16(x):
    u = jax.lax.bitcast_convert_type(x, jnp.uint32)
    r = (u + jnp.uint32(0x7FFF) + ((u >> 16) & jnp.uint32(1))) & jnp.uint32(0xFFFF0000)
    return jax.lax.bitcast_convert_type(r, jnp.float32)


def _occ_kernel(params_ref, off_ref, val_ref, occ_ref):
    w1t = params_ref[:, 0:3]
    b1 = params_ref[:, 3:4]
    w2 = params_ref[:, 4:5]
    b2 = params_ref[0:1, 5:6]

    v0 = pl.program_id(0) * _BLK
    lane = jax.lax.broadcasted_iota(jnp.int32, (1, _BLK), 1)
    v = v0 + lane
    ixf = (v >> 12).astype(jnp.float32)
    iyf = ((v >> 6) & (_RES - 1)).astype(jnp.float32)
    izf = (v & (_RES - 1)).astype(jnp.float32)
    scale = jnp.float32(2.0 / _RES)

    def sample_val(s):
        x = _bf16((ixf + off_ref[3 * s + 0:3 * s + 1, :]) * scale - 1.0)
        y = _bf16((iyf + off_ref[3 * s + 1:3 * s + 2, :]) * scale - 1.0)
        z = _bf16((izf + off_ref[3 * s + 2:3 * s + 3, :]) * scale - 1.0)
        p = jnp.concatenate([x, y, z], axis=0)
        a = jax.lax.dot_general(w1t, p, (((1,), (0,)), ((), ())),
                                preferred_element_type=jnp.float32)
        h = jnp.maximum(a + b1, 0.0)
        return jax.lax.dot_general(w2, _bf16(h), (((0,), (0,)), ((), ())),
                                   preferred_element_type=jnp.float32)

    occv = jnp.maximum(sample_val(0), sample_val(1)) + b2
    val_ref[...] = occv.reshape(1, 1, _BLK)
    occ_ref[...] = (occv > _OCC_THRE).reshape(1, 1, _BLK)


@jax.jit
def kernel(offsets, W1, b1, W2, b2):
    w1b = W1.astype(jnp.bfloat16).astype(jnp.float32)
    w2b = W2.astype(jnp.bfloat16).astype(jnp.float32)
    params = jnp.concatenate(
        [w1b.T,
         b1[:, None],
         w2b,
         jnp.full((_HID, 1), b2[0], dtype=jnp.float32)],
        axis=1,
    )

    off_t = offsets.reshape(_NVOX, 6).T

    val3, occ3 = pl.pallas_call(
        _occ_kernel,
        grid=(_GRID,),
        in_specs=[
            pl.BlockSpec((_HID, 6), lambda i: (0, 0)),
            pl.BlockSpec((6, _BLK), lambda i: (0, i)),
        ],
        out_specs=[
            pl.BlockSpec((1, 1, _BLK), lambda i: (i, 0, 0)),
            pl.BlockSpec((1, 1, _BLK), lambda i: (i, 0, 0)),
        ],
        out_shape=[
            jax.ShapeDtypeStruct((_GRID, 1, _BLK), jnp.float32),
            jax.ShapeDtypeStruct((_GRID, 1, _BLK), jnp.bool_),
        ],
    )(params, off_t)

    occ_val_grid = val3.reshape(_RES, _RES, _RES)
    occ_grid = occ3.reshape(_RES, _RES, _RES)
    return occ_grid, occ_val_grid


# speedup vs baseline: 10.8815x; 1.0506x over previous
import jax
import jax.numpy as jnp
from jax.experimental import pallas as pl

_RES = 64
_NVOX = _RES ** 3
_HID = 128
_OCC_THRE = 0.01

_BLK = 4096
_GRID = _NVOX // _BLK


def _bf16(x):
    u = jax.lax.bitcast_convert_type(x, jnp.uint32)
    r = (u + jnp.uint32(0x7FFF) + ((u >> 16) & jnp.uint32(1))) & jnp.uint32(0xFFFF0000)
    return jax.lax.bitcast_convert_type(r, jnp.float32)


def _occ_kernel(params_ref, off_ref, val_ref, occ_ref):
    w1t = params_ref[:, 0:3]
    b1 = params_ref[:, 3:4]
    w2 = params_ref[:, 4:5]
    b2 = params_ref[0:1, 5:6]

    v0 = pl.program_id(0) * _BLK
    lane = jax.lax.broadcasted_iota(jnp.int32, (1, _BLK), 1)
    v = v0 + lane
    ixf = (v >> 12).astype(jnp.float32)
    iyf = ((v >> 6) & (_RES - 1)).astype(jnp.float32)
    izf = (v & (_RES - 1)).astype(jnp.float32)
    scale = jnp.float32(2.0 / _RES)

    def coords(s):
        x = _bf16((ixf + off_ref[3 * s + 0:3 * s + 1, :]) * scale - 1.0)
        y = _bf16((iyf + off_ref[3 * s + 1:3 * s + 2, :]) * scale - 1.0)
        z = _bf16((izf + off_ref[3 * s + 2:3 * s + 3, :]) * scale - 1.0)
        return jnp.concatenate([x, y, z], axis=0)

    p = jnp.concatenate([coords(0), coords(1)], axis=1)
    a = jax.lax.dot_general(w1t, p, (((1,), (0,)), ((), ())),
                            preferred_element_type=jnp.float32)
    h = jnp.maximum(a + b1, 0.0)
    val = jax.lax.dot_general(w2, _bf16(h), (((0,), (0,)), ((), ())),
                              preferred_element_type=jnp.float32)
    occv = jnp.maximum(val[:, :_BLK], val[:, _BLK:]) + b2
    val_ref[...] = occv.reshape(1, 1, _BLK)
    occ_ref[...] = (occv > _OCC_THRE).reshape(1, 1, _BLK)


@jax.jit
def kernel(offsets, W1, b1, W2, b2):
    w1b = W1.astype(jnp.bfloat16).astype(jnp.float32)
    w2b = W2.astype(jnp.bfloat16).astype(jnp.float32)
    params = jnp.concatenate(
        [w1b.T,
         b1[:, None],
         w2b,
         jnp.full((_HID, 1), b2[0], dtype=jnp.float32)],
        axis=1,
    )

    off_t = offsets.reshape(_NVOX, 6).T

    val3, occ3 = pl.pallas_call(
        _occ_kernel,
        grid=(_GRID,),
        in_specs=[
            pl.BlockSpec((_HID, 6), lambda i: (0, 0)),
            pl.BlockSpec((6, _BLK), lambda i: (0, i)),
        ],
        out_specs=[
            pl.BlockSpec((1, 1, _BLK), lambda i: (i, 0, 0)),
            pl.BlockSpec((1, 1, _BLK), lambda i: (i, 0, 0)),
        ],
        out_shape=[
            jax.ShapeDtypeStruct((_GRID, 1, _BLK), jnp.float32),
            jax.ShapeDtypeStruct((_GRID, 1, _BLK), jnp.bool_),
        ],
    )(params, off_t)

    occ_val_grid = val3.reshape(_RES, _RES, _RES)
    occ_grid = occ3.reshape(_RES, _RES, _RES)
    return occ_grid, occ_val_grid
